# Initial kernel scaffold; baseline (speedup 1.0000x reference)
import jax
import jax.numpy as jnp
from jax.experimental import pallas as pl


def kernel(gains, bws, pms, gain_table, bw_table, pm_table, W_proj, b_proj, W_fc, b_fc):
    raise NotImplementedError("write your pallas kernel here")


# speedup vs baseline: 3.7243x; 3.7243x over previous
import functools

import jax
import jax.numpy as jnp
from jax import lax
from jax.experimental import pallas as pl
from jax.experimental.pallas import tpu as pltpu
from jax.experimental.pallas import tpu_sc as plsc

_B = 16384
_EMB = 128
_LAT = 64

@functools.lru_cache(maxsize=1)
def _make_gather3():
    info = plsc.get_sparse_core_info()
    nc, ns = info.num_cores, info.num_subcores
    nw = nc * ns
    bpw = _B // nw
    mesh = plsc.VectorSubcoreMesh(core_axis_name="c", subcore_axis_name="s")

    @functools.partial(
        pl.kernel,
        out_type=(jax.ShapeDtypeStruct((_B, _EMB), jnp.float32),) * 3,
        mesh=mesh,
        scratch_types=[
            pltpu.VMEM((bpw,), jnp.int32),
            pltpu.VMEM((bpw, _EMB), jnp.float32),
            pltpu.SemaphoreType.DMA,
        ],
    )
    def _gather3(g_hbm, b_hbm, p_hbm, gt_hbm, bt_hbm, pt_hbm,
                 og_hbm, ob_hbm, op_hbm, idx_v, rows_v, sem):
        wid = lax.axis_index("s") * nc + lax.axis_index("c")
        base = wid * bpw
        for idx_hbm, tbl_hbm, out_hbm in (
            (g_hbm, gt_hbm, og_hbm),
            (b_hbm, bt_hbm, ob_hbm),
            (p_hbm, pt_hbm, op_hbm),
        ):
            pltpu.sync_copy(idx_hbm.at[pl.ds(base, bpw)], idx_v)
            pltpu.async_copy(tbl_hbm.at[idx_v], rows_v, sem).wait()
            pltpu.sync_copy(rows_v, out_hbm.at[pl.ds(base, bpw)])

    return _gather3


_BS = 2048


def _proj_body(g_ref, b_ref, p_ref, wp_ref, bp_ref, wf_ref, bf_ref, o_ref):
    c = jnp.dot(g_ref[...], wp_ref[0:_EMB, :], preferred_element_type=jnp.float32)
    c += jnp.dot(b_ref[...], wp_ref[_EMB:2 * _EMB, :], preferred_element_type=jnp.float32)
    c += jnp.dot(p_ref[...], wp_ref[2 * _EMB:3 * _EMB, :], preferred_element_type=jnp.float32)
    c += bp_ref[...]
    o_ref[...] = jnp.dot(c, wf_ref[...], preferred_element_type=jnp.float32) + bf_ref[...]


def _project(g_embs, b_embs, p_embs, W_proj, b_proj, W_fc, b_fc):
    grid = (_B // _BS,)
    row_spec = pl.BlockSpec((_BS, _EMB), lambda i: (i, 0))
    full = lambda shape: pl.BlockSpec(shape, lambda i: (0,) * len(shape))
    return pl.pallas_call(
        _proj_body,
        grid=grid,
        in_specs=[
            row_spec, row_spec, row_spec,
            full((3 * _EMB, _EMB)),
            full((1, _EMB)),
            full((_EMB, _LAT)),
            full((1, _LAT)),
        ],
        out_specs=pl.BlockSpec((_BS, _LAT), lambda i: (i, 0)),
        out_shape=jax.ShapeDtypeStruct((_B, _LAT), jnp.float32),
    )(g_embs, b_embs, p_embs, W_proj, b_proj.reshape(1, _EMB),
      W_fc, b_fc.reshape(1, _LAT))


def kernel(gains, bws, pms, gain_table, bw_table, pm_table,
           W_proj, b_proj, W_fc, b_fc):
    gains = gains.astype(jnp.int32)
    bws = bws.astype(jnp.int32)
    pms = pms.astype(jnp.int32)
    g_embs, b_embs, p_embs = _make_gather3()(
        gains, bws, pms, gain_table, bw_table, pm_table)
    return _project(g_embs, b_embs, p_embs, W_proj, b_proj, W_fc, b_fc)
